# Initial kernel scaffold; baseline (speedup 1.0000x reference)
import jax
import jax.numpy as jnp
from jax.experimental import pallas as pl


def kernel(x, edge_index, W1l, W1r, b1, W2l, W2r, b2, Wc1, bc1, Wc2, bc2):
    raise NotImplementedError("write your pallas kernel here")


# speedup vs baseline: 6.4820x; 6.4820x over previous
import jax
import jax.numpy as jnp
from jax import lax
from jax.experimental import pallas as pl
from jax.experimental.pallas import tpu as pltpu
from jax.experimental.pallas import tpu_sc as plsc

N_NODES = 10000
N_EDGES = 320000
NP = 10008
C = 128
NW = 32
CPW = 79
EP = NW * CPW * C


def _make_sc_seg(D, with_deg):
    mesh = plsc.VectorSubcoreMesh(core_axis_name="c", subcore_axis_name="s")
    outs = [jax.ShapeDtypeStruct((2, NP, D), jnp.float32)]
    scratch = [
        pltpu.VMEM((C,), jnp.int32),
        pltpu.VMEM((C,), jnp.int32),
        pltpu.VMEM((C, D), jnp.float32),
        pltpu.VMEM_SHARED((NP, D), jnp.float32),
        pltpu.SemaphoreType.DMA,
    ]
    if with_deg:
        outs.append(jax.ShapeDtypeStruct((2, NP, 16), jnp.float32))
        scratch += [
            pltpu.VMEM((C, 16), jnp.float32),
            pltpu.VMEM_SHARED((NP, 16), jnp.float32),
        ]

    def body(*refs):
        if with_deg:
            (z, srcp, dstp, zacc, zdeg, ones, part, degpart,
             src_v, dst_v, rows_v, acc_sh, sem, ones_v, deg_sh) = refs
        else:
            (z, srcp, dstp, zacc, part,
             src_v, dst_v, rows_v, acc_sh, sem) = refs
        c = lax.axis_index("c")
        s = lax.axis_index("s")
        wid = s * 2 + c
        base = wid * (CPW * C)

        @pl.when(s == 0)
        def _zero():
            pltpu.sync_copy(zacc, acc_sh)
            if with_deg:
                pltpu.sync_copy(zdeg, deg_sh)

        if with_deg:
            pltpu.sync_copy(ones, ones_v)
        plsc.subcore_barrier()

        def chunk(i, carry):
            off = base + i * C
            pltpu.sync_copy(srcp.at[pl.ds(off, C)], src_v)
            pltpu.sync_copy(dstp.at[pl.ds(off, C)], dst_v)
            pltpu.async_copy(z.at[src_v], rows_v, sem).wait()
            pltpu.sync_copy(rows_v, acc_sh.at[dst_v], add=True)
            if with_deg:
                pltpu.sync_copy(ones_v, deg_sh.at[dst_v], add=True)
            return carry

        lax.fori_loop(0, CPW, chunk, 0)
        plsc.subcore_barrier()

        @pl.when(s == 0)
        def _writeback():
            pltpu.sync_copy(acc_sh, part.at[c])
            if with_deg:
                pltpu.sync_copy(deg_sh, degpart.at[c])

    return pl.kernel(body, out_type=tuple(outs), mesh=mesh,
                     scratch_types=tuple(scratch),
                     compiler_params=pltpu.CompilerParams(
                         use_tc_tiling_on_sc=False))


_B = 1000


def _tc1(x, wl, wr, b):
    def body(x_ref, wl_ref, wr_ref, b_ref, z_ref, y_ref):
        xb = x_ref[...]
        z_ref[...] = jnp.dot(xb, wl_ref[...], preferred_element_type=jnp.float32)
        y_ref[...] = (jnp.dot(xb, wr_ref[...], preferred_element_type=jnp.float32)
                      + b_ref[0:1, :])
    full = lambda i: (0, 0)
    row = lambda i: (i, 0)
    return pl.pallas_call(
        body,
        grid=(N_NODES // _B,),
        in_specs=[pl.BlockSpec((_B, 128), row), pl.BlockSpec((128, 64), full),
                  pl.BlockSpec((128, 64), full), pl.BlockSpec((8, 64), full)],
        out_specs=[pl.BlockSpec((_B, 64), row), pl.BlockSpec((_B, 64), row)],
        out_shape=[jax.ShapeDtypeStruct((N_NODES, 64), jnp.float32)] * 2,
    )(x, wl, wr, b)


def _tc2(p0, p1, d0, d1, y1, wl, wr, b):
    def body(p0_ref, p1_ref, d0_ref, d1_ref, y1_ref, wl_ref, wr_ref, b_ref,
             z_ref, y_ref):
        deg = d0_ref[...][:, 0:1] + d1_ref[...][:, 0:1]
        degc = jnp.maximum(deg, 1.0)
        h1 = jnp.maximum((p0_ref[...] + p1_ref[...]) / degc + y1_ref[...], 0.0)
        z_ref[...] = jnp.dot(h1, wl_ref[...], preferred_element_type=jnp.float32)
        y_ref[...] = (jnp.dot(h1, wr_ref[...], preferred_element_type=jnp.float32)
                      + b_ref[0:1, :])
    full = lambda i: (0, 0)
    row = lambda i: (i, 0)
    return pl.pallas_call(
        body,
        grid=(N_NODES // _B,),
        in_specs=[pl.BlockSpec((_B, 64), row), pl.BlockSpec((_B, 64), row),
                  pl.BlockSpec((_B, 16), row), pl.BlockSpec((_B, 16), row),
                  pl.BlockSpec((_B, 64), row), pl.BlockSpec((64, 32), full),
                  pl.BlockSpec((64, 32), full), pl.BlockSpec((8, 32), full)],
        out_specs=[pl.BlockSpec((_B, 32), row), pl.BlockSpec((_B, 32), row)],
        out_shape=[jax.ShapeDtypeStruct((N_NODES, 32), jnp.float32)] * 2,
    )(p0, p1, d0, d1, y1, wl, wr, b)


def _tc3(q0, q1, d0, d1, y2, wc1, b1, wc2, b2):
    def body(q0_ref, q1_ref, d0_ref, d1_ref, y2_ref, wc1_ref, b1_ref,
             wc2_ref, b2_ref, out_ref):
        deg = d0_ref[...][:, 0:1] + d1_ref[...][:, 0:1]
        degc = jnp.maximum(deg, 1.0)
        h2 = jnp.maximum((q0_ref[...] + q1_ref[...]) / degc + y2_ref[...], 0.0)
        c1 = jnp.maximum(
            jnp.dot(h2, wc1_ref[...], preferred_element_type=jnp.float32)
            + b1_ref[0:1, :], 0.0)
        out_ref[...] = (jnp.dot(c1, wc2_ref[...], preferred_element_type=jnp.float32)
                        + b2_ref[0:1, :])
    full = lambda i: (0, 0)
    row = lambda i: (i, 0)
    return pl.pallas_call(
        body,
        grid=(N_NODES // _B,),
        in_specs=[pl.BlockSpec((_B, 32), row), pl.BlockSpec((_B, 32), row),
                  pl.BlockSpec((_B, 16), row), pl.BlockSpec((_B, 16), row),
                  pl.BlockSpec((_B, 32), row), pl.BlockSpec((32, 16), full),
                  pl.BlockSpec((8, 16), full), pl.BlockSpec((16, 2), full),
                  pl.BlockSpec((8, 2), full)],
        out_specs=pl.BlockSpec((_B, 2), row),
        out_shape=jax.ShapeDtypeStruct((N_NODES, 2), jnp.float32),
    )(q0, q1, d0, d1, y2, wc1, b1, wc2, b2)


def kernel(x, edge_index, W1l, W1r, b1, W2l, W2r, b2, Wc1, bc1, Wc2, bc2):
    f32 = jnp.float32
    src = edge_index[0].astype(jnp.int32)
    dst = edge_index[1].astype(jnp.int32)
    pad = EP - N_EDGES
    srcp = jnp.concatenate([src, jnp.zeros((pad,), jnp.int32)])
    dstp = jnp.concatenate([dst, jnp.full((pad,), N_NODES, jnp.int32)])

    zacc64 = jnp.zeros((NP, 64), f32)
    zacc32 = jnp.zeros((NP, 32), f32)
    zdeg = jnp.zeros((NP, 16), f32)
    ones = jnp.ones((C, 16), f32)

    z1, y1 = _tc1(x, W1l.T, W1r.T, jnp.broadcast_to(b1, (8, 64)))

    part1, degp = _make_sc_seg(64, True)(z1, srcp, dstp, zacc64, zdeg, ones)
    p0 = part1[0, :N_NODES]
    p1 = part1[1, :N_NODES]
    d0 = degp[0, :N_NODES]
    d1 = degp[1, :N_NODES]

    z2, y2 = _tc2(p0, p1, d0, d1, y1, W2l.T, W2r.T,
                  jnp.broadcast_to(b2, (8, 32)))

    (part2,) = _make_sc_seg(32, False)(z2, srcp, dstp, zacc32)

    out = _tc3(part2[0, :N_NODES], part2[1, :N_NODES], d0, d1, y2,
               Wc1.T, jnp.broadcast_to(bc1, (8, 16)),
               Wc2.T, jnp.broadcast_to(bc2, (8, 2)))
    return out


# speedup vs baseline: 7.7204x; 1.1911x over previous
import jax
import jax.numpy as jnp
from jax import lax
from jax.experimental import pallas as pl
from jax.experimental.pallas import tpu as pltpu
from jax.experimental.pallas import tpu_sc as plsc

N_NODES = 10000
N_EDGES = 320000
NP = 10008
C = 128
NW = 32
CPW = 80
EP = NW * CPW * C
NBUF = 4


def _make_sc_seg(D, with_deg):
    mesh = plsc.VectorSubcoreMesh(core_axis_name="c", subcore_axis_name="s")
    outs = [jax.ShapeDtypeStruct((2, NP, D), jnp.float32)]
    scratch = [
        pltpu.VMEM((CPW, C), jnp.int32),
        pltpu.VMEM((CPW, C), jnp.int32),
        pltpu.VMEM_SHARED((NP, D), jnp.float32),
    ]
    scratch += [pltpu.VMEM((C, D), jnp.float32) for _ in range(NBUF)]
    scratch += [pltpu.SemaphoreType.DMA for _ in range(NBUF)]
    if with_deg:
        outs.append(jax.ShapeDtypeStruct((2, NP, 16), jnp.float32))
        scratch += [
            pltpu.VMEM((C, 16), jnp.float32),
            pltpu.VMEM_SHARED((NP, 16), jnp.float32),
        ]

    def body(*refs):
        if with_deg:
            (z, srcp, dstp, zacc, zdeg, ones, part, degpart,
             src_all, dst_all, acc_sh, *rest) = refs
            rows = rest[:NBUF]
            sems = rest[NBUF:2 * NBUF]
            ones_v, deg_sh = rest[2 * NBUF:]
        else:
            (z, srcp, dstp, zacc, part,
             src_all, dst_all, acc_sh, *rest) = refs
            rows = rest[:NBUF]
            sems = rest[NBUF:2 * NBUF]
        c = lax.axis_index("c")
        s = lax.axis_index("s")
        wid = s * 2 + c

        @pl.when(s == 0)
        def _zero():
            pltpu.sync_copy(zacc, acc_sh)
            if with_deg:
                pltpu.sync_copy(zdeg, deg_sh)

        pltpu.sync_copy(srcp.at[wid], src_all)
        pltpu.sync_copy(dstp.at[wid], dst_all)
        if with_deg:
            pltpu.sync_copy(ones, ones_v)
        plsc.subcore_barrier()

        for b in range(NBUF):
            pltpu.async_copy(z.at[src_all.at[b]], rows[b], sems[b])

        def group(g, carry):
            for b in range(NBUF):
                i = g * NBUF + b
                pltpu.make_async_copy(z.at[src_all.at[i]], rows[b],
                                      sems[b]).wait()
                pltpu.sync_copy(rows[b], acc_sh.at[dst_all.at[i]], add=True)
                if with_deg:
                    pltpu.sync_copy(ones_v, deg_sh.at[dst_all.at[i]],
                                    add=True)
                nxt = i + NBUF

                @pl.when(nxt < CPW)
                def _prefetch():
                    pltpu.async_copy(z.at[src_all.at[nxt]], rows[b], sems[b])
            return carry

        lax.fori_loop(0, CPW // NBUF, group, 0)
        plsc.subcore_barrier()

        @pl.when(s == 0)
        def _writeback():
            pltpu.sync_copy(acc_sh, part.at[c])
            if with_deg:
                pltpu.sync_copy(deg_sh, degpart.at[c])

    return pl.kernel(body, out_type=tuple(outs), mesh=mesh,
                     scratch_types=tuple(scratch),
                     compiler_params=pltpu.CompilerParams(
                         use_tc_tiling_on_sc=False))


_B = 1000


def _tc1(x, wl, wr, b):
    def body(x_ref, wl_ref, wr_ref, b_ref, z_ref, y_ref):
        xb = x_ref[...]
        z_ref[...] = jnp.dot(xb, wl_ref[...], preferred_element_type=jnp.float32)
        y_ref[...] = (jnp.dot(xb, wr_ref[...], preferred_element_type=jnp.float32)
                      + b_ref[0:1, :])
    full = lambda i: (0, 0)
    row = lambda i: (i, 0)
    return pl.pallas_call(
        body,
        grid=(N_NODES // _B,),
        in_specs=[pl.BlockSpec((_B, 128), row), pl.BlockSpec((128, 64), full),
                  pl.BlockSpec((128, 64), full), pl.BlockSpec((8, 64), full)],
        out_specs=[pl.BlockSpec((_B, 64), row), pl.BlockSpec((_B, 64), row)],
        out_shape=[jax.ShapeDtypeStruct((N_NODES, 64), jnp.float32)] * 2,
    )(x, wl, wr, b)


def _tc2(p0, p1, d0, d1, y1, wl, wr, b):
    def body(p0_ref, p1_ref, d0_ref, d1_ref, y1_ref, wl_ref, wr_ref, b_ref,
             z_ref, y_ref):
        deg = d0_ref[...][:, 0:1] + d1_ref[...][:, 0:1]
        degc = jnp.maximum(deg, 1.0)
        h1 = jnp.maximum((p0_ref[...] + p1_ref[...]) / degc + y1_ref[...], 0.0)
        z_ref[...] = jnp.dot(h1, wl_ref[...], preferred_element_type=jnp.float32)
        y_ref[...] = (jnp.dot(h1, wr_ref[...], preferred_element_type=jnp.float32)
                      + b_ref[0:1, :])
    full = lambda i: (0, 0)
    row = lambda i: (i, 0)
    return pl.pallas_call(
        body,
        grid=(N_NODES // _B,),
        in_specs=[pl.BlockSpec((_B, 64), row), pl.BlockSpec((_B, 64), row),
                  pl.BlockSpec((_B, 16), row), pl.BlockSpec((_B, 16), row),
                  pl.BlockSpec((_B, 64), row), pl.BlockSpec((64, 32), full),
                  pl.BlockSpec((64, 32), full), pl.BlockSpec((8, 32), full)],
        out_specs=[pl.BlockSpec((_B, 32), row), pl.BlockSpec((_B, 32), row)],
        out_shape=[jax.ShapeDtypeStruct((N_NODES, 32), jnp.float32)] * 2,
    )(p0, p1, d0, d1, y1, wl, wr, b)


def _tc3(q0, q1, d0, d1, y2, wc1, b1, wc2, b2):
    def body(q0_ref, q1_ref, d0_ref, d1_ref, y2_ref, wc1_ref, b1_ref,
             wc2_ref, b2_ref, out_ref):
        deg = d0_ref[...][:, 0:1] + d1_ref[...][:, 0:1]
        degc = jnp.maximum(deg, 1.0)
        h2 = jnp.maximum((q0_ref[...] + q1_ref[...]) / degc + y2_ref[...], 0.0)
        c1 = jnp.maximum(
            jnp.dot(h2, wc1_ref[...], preferred_element_type=jnp.float32)
            + b1_ref[0:1, :], 0.0)
        out_ref[...] = (jnp.dot(c1, wc2_ref[...], preferred_element_type=jnp.float32)
                        + b2_ref[0:1, :])
    full = lambda i: (0, 0)
    row = lambda i: (i, 0)
    return pl.pallas_call(
        body,
        grid=(N_NODES // _B,),
        in_specs=[pl.BlockSpec((_B, 32), row), pl.BlockSpec((_B, 32), row),
                  pl.BlockSpec((_B, 16), row), pl.BlockSpec((_B, 16), row),
                  pl.BlockSpec((_B, 32), row), pl.BlockSpec((32, 16), full),
                  pl.BlockSpec((8, 16), full), pl.BlockSpec((16, 2), full),
                  pl.BlockSpec((8, 2), full)],
        out_specs=pl.BlockSpec((_B, 2), row),
        out_shape=jax.ShapeDtypeStruct((N_NODES, 2), jnp.float32),
    )(q0, q1, d0, d1, y2, wc1, b1, wc2, b2)


def kernel(x, edge_index, W1l, W1r, b1, W2l, W2r, b2, Wc1, bc1, Wc2, bc2):
    f32 = jnp.float32
    src = edge_index[0].astype(jnp.int32)
    dst = edge_index[1].astype(jnp.int32)
    pad = EP - N_EDGES
    srcp = jnp.concatenate([src, jnp.zeros((pad,), jnp.int32)]
                           ).reshape(NW, CPW, C)
    dstp = jnp.concatenate([dst, jnp.full((pad,), N_NODES, jnp.int32)]
                           ).reshape(NW, CPW, C)

    zacc64 = jnp.zeros((NP, 64), f32)
    zacc32 = jnp.zeros((NP, 32), f32)
    zdeg = jnp.zeros((NP, 16), f32)
    ones = jnp.ones((C, 16), f32)

    z1, y1 = _tc1(x, W1l.T, W1r.T, jnp.broadcast_to(b1, (8, 64)))

    part1, degp = _make_sc_seg(64, True)(z1, srcp, dstp, zacc64, zdeg, ones)
    p0 = part1[0, :N_NODES]
    p1 = part1[1, :N_NODES]
    d0 = degp[0, :N_NODES]
    d1 = degp[1, :N_NODES]

    z2, y2 = _tc2(p0, p1, d0, d1, y1, W2l.T, W2r.T,
                  jnp.broadcast_to(b2, (8, 32)))

    (part2,) = _make_sc_seg(32, False)(z2, srcp, dstp, zacc32)

    out = _tc3(part2[0, :N_NODES], part2[1, :N_NODES], d0, d1, y2,
               Wc1.T, jnp.broadcast_to(bc1, (8, 16)),
               Wc2.T, jnp.broadcast_to(bc2, (8, 2)))
    return out
